# baseline (speedup 1.0000x reference)
import functools

import jax
import jax.numpy as jnp
from jax import lax
from jax.experimental import pallas as pl
from jax.experimental.pallas import tpu as pltpu
from jax.experimental.pallas import tpu_sc as plsc

BATCH = 16384
N_DIM = 64
N_USERS = 1000000
L = 16
NC, NS = 2, 16
NW = NC * NS
PER_W = BATCH // NW
SUPER = 1024
SUB = 128
SPS = SUPER // SUB
NSUPER = N_USERS // SUPER
TAIL_LO = NSUPER * SUPER
TAIL_W = NSUPER % NW
TAIL_N = N_USERS - TAIL_LO
HITCAP = 768
HGRP = HITCAP // L
UCHUNK = 2048
DUMMY = BATCH
HUGEU = 1 << 30


def _wid():
    return lax.axis_index("s") * NC + lax.axis_index("c")


def _popcount(m):
    return plsc.all_reduce_population_count(m)[0]


def _main_body(u_hbm, uvt_hbm, iv_hbm, ip_hbm, ap_hbm, ub_hbm, ib_hbm,
               c1_hbm, c0_hbm, ids_out, vals_out,
               u_ch, hit_ids, hit_us, chid, chu, ids_sw, us_sw,
               buf0, buf1, vu_hit, i_v, a_v, ubv, ibv, vals_v,
               c1_v, c0_v, ibf0, ibf1, posm,
               bsem, sems):
    bufs = [buf0, buf1]
    ibf = [ibf0, ibf1]
    wid = _wid()
    lane = jnp.arange(L, dtype=jnp.int32)

    def fill(t, c):
        sl = pl.ds(t * L, L)
        hit_ids[sl] = jnp.full((L,), DUMMY, jnp.int32)
        hit_us[sl] = jnp.full((L,), HUGEU, jnp.int32)
        ids_sw[sl] = jnp.full((L,), DUMMY, jnp.int32)
        us_sw[sl] = jnp.zeros((L,), jnp.int32)
        return c

    lax.fori_loop(0, HGRP, fill, 0)

    pos = jnp.int32(0)
    for c in range(BATCH // UCHUNK):
        pltpu.sync_copy(u_hbm.at[pl.ds(c * UCHUNK, UCHUNK)], u_ch)

        def scan(t, p, c=c):
            uv = u_ch[pl.ds(t * L, L)]
            m = ((uv >> 10) & (NW - 1)) == wid
            plsc.store_compressed(hit_ids.at[pl.ds(p, L)],
                                  lane + (c * UCHUNK + t * L), mask=m)
            plsc.store_compressed(hit_us.at[pl.ds(p, L)], uv, mask=m)
            return p + _popcount(m)

        pos = lax.fori_loop(0, UCHUNK // L, scan, pos)

    def fetch_at(coff, par, width):
        coff = pl.multiple_of(coff, 128)
        for g in range(8):
            pltpu.async_copy(
                uvt_hbm.at[pl.ds(8 * g, 8), pl.ds(coff, width)],
                bufs[par].at[pl.ds(8 * g, 8), pl.ds(0, width)],
                sems.at[par])

    def drain(par, width):
        for g in range(8):
            pltpu.make_async_copy(
                uvt_hbm.at[pl.ds(0, 8), pl.ds(0, width)],
                bufs[par].at[pl.ds(8 * g, 8), pl.ds(0, width)],
                sems.at[par]).wait()

    def sub_lo(sc):
        return (wid + NW * (sc // SPS)) * SUPER + (sc % SPS) * SUB

    def extract(lo, width, par, pos2):
        def refil(t, cpos):
            sl = pl.ds(t * L, L)
            uh = hit_us[sl]
            m = (uh >= lo) & (uh < lo + width)
            plsc.store_compressed(chu.at[pl.ds(cpos, L)], uh, mask=m)
            plsc.store_compressed(chid.at[pl.ds(cpos, L)], hit_ids[sl],
                                  mask=m)
            return cpos + _popcount(m)

        cnt = lax.fori_loop(0, HGRP, refil, 0)

        def xgrp(t, p):
            m = lane < (cnt - t * L)
            sl = pl.ds(t * L, L)
            uoff = chu[sl] - lo
            rows = p + lane
            plsc.store_scatter(ids_sw, [rows], chid[sl], mask=m)
            plsc.store_scatter(us_sw, [rows], chu[sl], mask=m)
            dc = jnp.zeros((L,), jnp.int32)
            rbase = rows << 6
            for _ in range(N_DIM):
                xd = plsc.load_gather(bufs[par], [dc, uoff], mask=m)
                plsc.store_scatter(vu_hit, [rbase + dc], xd, mask=m)
                dc = dc + 1
            return p + _popcount(m)

        ng = (cnt + L - 1) >> 4
        return lax.fori_loop(0, ng, xgrp, pos2)

    nsub = jnp.where(wid < (NSUPER % NW), SPS * (NSUPER // NW + 1),
                     SPS * (NSUPER // NW)).astype(jnp.int32)

    fetch_at(sub_lo(0), 0, SUB)
    fetch_at(sub_lo(1), 1, SUB)

    def pair(p, pos2):
        sc0 = 2 * p
        drain(0, SUB)
        pos2 = extract(sub_lo(sc0), SUB, 0, pos2)

        @pl.when(sc0 + 2 < nsub)
        def _():
            fetch_at(sub_lo(sc0 + 2), 0, SUB)

        sc1 = sc0 + 1
        drain(1, SUB)
        pos2 = extract(sub_lo(sc1), SUB, 1, pos2)

        @pl.when(sc1 + 2 < nsub)
        def _():
            fetch_at(sub_lo(sc1 + 2), 1, SUB)

        return pos2

    pos2 = lax.fori_loop(0, nsub // 2, pair, jnp.int32(0))
    posm[0] = pos2

    @pl.when(wid == TAIL_W)
    def _():
        p = posm[0]
        for t in range(TAIL_N // SUB):
            fetch_at(TAIL_LO + t * SUB, 0, SUB)
            drain(0, SUB)
            p = extract(TAIL_LO + t * SUB, SUB, 0, p)
        fetch_at(TAIL_LO + 4 * SUB, 0, 64)
        drain(0, 64)
        p = extract(TAIL_LO + 4 * SUB, 64, 0, p)
        posm[0] = p

    d1 = [pltpu.async_copy(t_hbm.at[ids_sw.at[pl.ds(c * 128, 128)]],
                           t_v.at[pl.ds(c * 128, 128)], bsem)
          for c in range(HITCAP // 128)
          for (t_hbm, t_v) in ((ip_hbm, i_v), (ap_hbm, a_v))]
    d1.append(pltpu.async_copy(c1_hbm, c1_v, bsem))
    d1.append(pltpu.async_copy(c0_hbm, c0_v, bsem))
    for d in d1:
        d.wait()
    d2 = [pltpu.async_copy(t_hbm.at[t_idx.at[pl.ds(c * 128, 128)]],
                           t_v.at[pl.ds(c * 128, 128)], bsem)
          for c in range(HITCAP // 128)
          for (t_hbm, t_idx, t_v) in ((ub_hbm, us_sw, ubv),
                                      (ib_hbm, i_v, ibv))]
    for d in d2:
        d.wait()

    c1v = c1_v[...]
    c0v = c0_v[...]

    def fire(g, par):
        ivals = i_v[pl.ds(g * L, L)]
        for k in range(L):
            is_ = pl.multiple_of((ivals[k] >> 3) << 3, 8)
            pltpu.async_copy(iv_hbm.at[pl.ds(is_, 8), :], ibf[par].at[k],
                             sems.at[par])

    def idrain(par):
        for k in range(L):
            pltpu.make_async_copy(iv_hbm.at[pl.ds(0, 8), :],
                                  ibf[par].at[k], sems.at[par]).wait()

    def compute(g, par):
        sl = pl.ds(g * L, L)
        isub = i_v[sl] & 7
        rows = lane + g * L
        accs = [a_v[sl] * c1v + c0v, ubv[sl] + ibv[sl],
                jnp.zeros((L,), jnp.float32), jnp.zeros((L,), jnp.float32)]
        dc = jnp.zeros((L,), jnp.int32)
        rbase = rows << 6
        for d in range(N_DIM):
            xu = plsc.load_gather(vu_hit, [rbase + dc])
            xi = plsc.load_gather(ibf[par], [lane, isub, dc])
            accs[d % 4] = accs[d % 4] + xu * xi
            dc = dc + 1
        vals_v[sl] = (accs[0] + accs[1]) + (accs[2] + accs[3])

    fire(0, 0)
    fire(1, 1)

    def gpair(p, carry):
        g0 = 2 * p
        idrain(0)
        compute(g0, 0)

        @pl.when(g0 + 2 < HGRP)
        def _():
            fire(g0 + 2, 0)

        idrain(1)
        compute(g0 + 1, 1)

        @pl.when(g0 + 3 < HGRP)
        def _():
            fire(g0 + 3, 1)

        return carry

    lax.fori_loop(0, HGRP // 2, gpair, 0)
    pltpu.sync_copy(ids_sw, ids_out.at[wid])
    pltpu.sync_copy(vals_v, vals_out.at[wid])


def _redist_body(ids_hbm, vals_hbm, out_hbm, ids_all, vals_all, out_v):
    wid = _wid()
    pltpu.sync_copy(ids_hbm, ids_all)
    pltpu.sync_copy(vals_hbm, vals_all)

    def w_iter(w, c):
        def q_iter(q, c2):
            sl = pl.ds(q * L, L)
            ids16 = ids_all[w, sl]
            m = (ids16 >> 9) == wid
            plsc.store_scatter(out_v, [ids16 & (PER_W - 1)],
                               vals_all[w, sl], mask=m)
            return c2

        return lax.fori_loop(0, HGRP, q_iter, c)

    lax.fori_loop(0, NW, w_iter, 0)
    pltpu.sync_copy(out_v, out_hbm.at[pl.ds(wid * PER_W, PER_W)])


def _mesh():
    return plsc.VectorSubcoreMesh(core_axis_name="c", subcore_axis_name="s")


@jax.jit
def _mfpoly2_sc(u1, ip, ap, user_vect_t, ub_flat, item_vect, ib_flat,
                c1, c0):
    cp = pltpu.CompilerParams(needs_layout_passes=False)
    f32, i32 = jnp.float32, jnp.int32
    main = functools.partial(
        pl.kernel, mesh=_mesh(), compiler_params=cp,
        out_type=(jax.ShapeDtypeStruct((NW, HITCAP), i32),
                  jax.ShapeDtypeStruct((NW, HITCAP), f32)),
        scratch_types=[
            pltpu.VMEM((UCHUNK,), i32),
            pltpu.VMEM((HITCAP,), i32),
            pltpu.VMEM((HITCAP,), i32),
            pltpu.VMEM((HITCAP,), i32),
            pltpu.VMEM((HITCAP,), i32),
            pltpu.VMEM((HITCAP,), i32),
            pltpu.VMEM((HITCAP,), i32),
            pltpu.VMEM((N_DIM, SUB), f32),
            pltpu.VMEM((N_DIM, SUB), f32),
            pltpu.VMEM((HITCAP * N_DIM,), f32),
            pltpu.VMEM((HITCAP,), i32),
            pltpu.VMEM((HITCAP,), f32),
            pltpu.VMEM((HITCAP,), f32),
            pltpu.VMEM((HITCAP,), f32),
            pltpu.VMEM((HITCAP,), f32),
            pltpu.VMEM((L,), f32),
            pltpu.VMEM((L,), f32),
            pltpu.VMEM((L, 8, N_DIM), f32),
            pltpu.VMEM((L, 8, N_DIM), f32),
            pltpu.SMEM((1,), i32),
            pltpu.SemaphoreType.DMA,
            pltpu.SemaphoreType.DMA((2,)),
        ])(_main_body)
    ids, vals = main(u1, user_vect_t, item_vect, ip, ap, ub_flat, ib_flat,
                     c1, c0)

    redist = functools.partial(
        pl.kernel, mesh=_mesh(), compiler_params=cp,
        out_type=jax.ShapeDtypeStruct((BATCH,), f32),
        scratch_types=[
            pltpu.VMEM((NW, HITCAP), i32),
            pltpu.VMEM((NW, HITCAP), f32),
            pltpu.VMEM((PER_W,), f32),
        ])(_redist_body)
    return redist(ids, vals)


def kernel(u, i, a, user_vect, user_bias, item_vect, item_bias, glob_bias,
           age1_w, age1_b, age2_w, age2_b):
    n = u.shape[0]
    u32 = u.astype(jnp.int32)
    i32_ = i.astype(jnp.int32)
    ip = jnp.pad(i32_, (0, L))
    ap = jnp.pad(a, (0, L))
    c1 = age1_w[0, 0] * age2_w[0, 0]
    c0 = glob_bias[0, 0] + age1_b[0] * age2_w[0, 0] + age2_b[0]
    c1v = jnp.full((L,), c1, jnp.float32)
    c0v = jnp.full((L,), c0, jnp.float32)
    out = _mfpoly2_sc(u32, ip, ap, user_vect.T, user_bias.reshape(-1),
                      item_vect, item_bias.reshape(-1), c1v, c0v)
    return out.reshape(n)


# speedup vs baseline: 1.0697x; 1.0697x over previous
import functools

import jax
import jax.numpy as jnp
from jax import lax
from jax.experimental import pallas as pl
from jax.experimental.pallas import tpu as pltpu
from jax.experimental.pallas import tpu_sc as plsc

BATCH = 16384
N_DIM = 64
N_USERS = 1000000
L = 16
NC, NS = 2, 16
NW = NC * NS
PER_W = BATCH // NW
WIN = 1024
NWIN = N_USERS // WIN
TAIL_LO = NWIN * WIN
TAIL_W = NWIN % NW
TAIL_N = N_USERS - TAIL_LO
HITCAP = 768
HGRP = HITCAP // L
WCAP = 96
UCHUNK = 2048
DUMMY = BATCH
HUGEU = 1 << 30


def _wid():
    return lax.axis_index("s") * NC + lax.axis_index("c")


def _popcount(m):
    return plsc.all_reduce_population_count(m)[0]


def _main_body(u_hbm, uvt_hbm, iv_hbm, ip_hbm, ap_hbm, ub_hbm, ib_hbm,
               c1_hbm, c0_hbm, ids_out, vals_out,
               u_ch, hit_ids, hit_us, chid, chu, ids_sw, us_sw,
               buf0, buf1, tbuf, vu_hit, i_v, a_v, ubv, ibv, vals_v,
               c1_v, c0_v, ibf0, ibf1, posm,
               bsem, sems):
    bufs = [buf0, buf1]
    ibf = [ibf0, ibf1]
    wid = _wid()
    lane = jnp.arange(L, dtype=jnp.int32)

    def fill(t, c):
        sl = pl.ds(t * L, L)
        hit_ids[sl] = jnp.full((L,), DUMMY, jnp.int32)
        hit_us[sl] = jnp.full((L,), HUGEU, jnp.int32)
        ids_sw[sl] = jnp.full((L,), DUMMY, jnp.int32)
        us_sw[sl] = jnp.zeros((L,), jnp.int32)
        return c

    lax.fori_loop(0, HGRP, fill, 0)

    pos = jnp.int32(0)
    for c in range(BATCH // UCHUNK):
        pltpu.sync_copy(u_hbm.at[pl.ds(c * UCHUNK, UCHUNK)], u_ch)

        def scan(t, p, c=c):
            uv = u_ch[pl.ds(t * L, L)]
            m = ((uv >> 10) & (NW - 1)) == wid
            plsc.store_compressed(hit_ids.at[pl.ds(p, L)],
                                  lane + (c * UCHUNK + t * L), mask=m)
            plsc.store_compressed(hit_us.at[pl.ds(p, L)], uv, mask=m)
            return p + _popcount(m)

        pos = lax.fori_loop(0, UCHUNK // L, scan, pos, unroll=2)

    def fetch(lo, g, par, width):
        coff = pl.multiple_of(lo, 128)
        pltpu.async_copy(
            uvt_hbm.at[pl.ds(8 * g, 8), pl.ds(coff, width)],
            bufs[par].at[:, pl.ds(0, width)],
            sems.at[par])

    def drain(par, width):
        pltpu.make_async_copy(
            uvt_hbm.at[pl.ds(0, 8), pl.ds(0, width)],
            bufs[par].at[:, pl.ds(0, width)],
            sems.at[par]).wait()

    def win_lo(w):
        return (wid + NW * w) * WIN

    def refill(lo, width):
        def rf(t, cpos):
            sl = pl.ds(t * L, L)
            uh = hit_us[sl]
            m = (uh >= lo) & (uh < lo + width)
            plsc.store_compressed(chu.at[pl.ds(cpos, L)], uh, mask=m)
            plsc.store_compressed(chid.at[pl.ds(cpos, L)], hit_ids[sl],
                                  mask=m)
            return cpos + _popcount(m)

        cnt = lax.fori_loop(0, HGRP, rf, 0, unroll=2)
        p2 = posm[0]

        def sw(t, c):
            m = lane < (cnt - t * L)
            sl = pl.ds(t * L, L)
            rows = p2 + t * L + lane
            plsc.store_scatter(ids_sw, [rows], chid[sl], mask=m)
            plsc.store_scatter(us_sw, [rows], chu[sl], mask=m)
            return c

        lax.fori_loop(0, (cnt + L - 1) >> 4, sw, 0)
        return cnt

    def extract(buf, lo, g, cnt):
        p2 = posm[0]

        def xgrp(t, c):
            m = lane < (cnt - t * L)
            sl = pl.ds(t * L, L)
            uoff = chu[sl] - lo
            rbase = (p2 + t * L + lane) << 6
            dl = jnp.full((L,), 0, jnp.int32)
            dg = jnp.full((L,), 8 * g, jnp.int32)
            for _ in range(8):
                xd = plsc.load_gather(buf, [dl, uoff], mask=m)
                plsc.store_scatter(vu_hit, [rbase + dg], xd, mask=m)
                dl = dl + 1
                dg = dg + 1
            return c

        lax.fori_loop(0, (cnt + L - 1) >> 4, xgrp, 0)

    nwin = jnp.where(wid < (NWIN % NW), NWIN // NW + 1,
                     NWIN // NW).astype(jnp.int32)
    posm[0] = 0

    fetch(win_lo(0), 0, 0, WIN)

    def wloop(w, carry):
        cnt = refill(win_lo(w), WIN)
        for g in range(8):
            par = g % 2
            if g < 7:
                fetch(win_lo(w), g + 1, 1 - par, WIN)
            else:
                @pl.when(w + 1 < nwin)
                def _():
                    fetch(win_lo(w + 1), 0, 1 - par, WIN)
            drain(par, WIN)
            extract(bufs[par], win_lo(w), g, cnt)
        posm[0] = posm[0] + cnt
        return carry

    lax.fori_loop(0, nwin, wloop, 0)

    @pl.when(wid == TAIL_W)
    def _():
        cnt = refill(TAIL_LO, 512)
        for g in range(8):
            par = g % 2
            pltpu.async_copy(
                uvt_hbm.at[pl.ds(8 * g, 8), pl.ds(TAIL_LO, 512)],
                bufs[par].at[:, pl.ds(0, 512)], sems.at[par])
            drain(par, 512)
            extract(bufs[par], TAIL_LO, g, cnt)
        posm[0] = posm[0] + cnt
        cnt2 = refill(TAIL_LO + 512, 64)
        for g in range(8):
            pltpu.async_copy(
                uvt_hbm.at[pl.ds(8 * g, 8), pl.ds(TAIL_LO + 512, 64)],
                tbuf, sems.at[0])
            pltpu.make_async_copy(
                uvt_hbm.at[pl.ds(8 * g, 8), pl.ds(TAIL_LO + 512, 64)],
                tbuf, sems.at[0]).wait()
            extract(tbuf, TAIL_LO + 512, g, cnt2)
        posm[0] = posm[0] + cnt2

    d1 = [pltpu.async_copy(t_hbm.at[ids_sw.at[pl.ds(c * 128, 128)]],
                           t_v.at[pl.ds(c * 128, 128)], bsem)
          for c in range(HITCAP // 128)
          for (t_hbm, t_v) in ((ip_hbm, i_v), (ap_hbm, a_v))]
    d1.append(pltpu.async_copy(c1_hbm, c1_v, bsem))
    d1.append(pltpu.async_copy(c0_hbm, c0_v, bsem))
    for d in d1:
        d.wait()
    d2 = [pltpu.async_copy(t_hbm.at[t_idx.at[pl.ds(c * 128, 128)]],
                           t_v.at[pl.ds(c * 128, 128)], bsem)
          for c in range(HITCAP // 128)
          for (t_hbm, t_idx, t_v) in ((ub_hbm, us_sw, ubv),
                                      (ib_hbm, i_v, ibv))]
    for d in d2:
        d.wait()

    c1v = c1_v[...]
    c0v = c0_v[...]

    def fire(g, par):
        ivals = i_v[pl.ds(g * L, L)]
        for k in range(L):
            is_ = pl.multiple_of((ivals[k] >> 3) << 3, 8)
            pltpu.async_copy(iv_hbm.at[pl.ds(is_, 8), :], ibf[par].at[k],
                             sems.at[par])

    def idrain(par):
        for k in range(L):
            pltpu.make_async_copy(iv_hbm.at[pl.ds(0, 8), :],
                                  ibf[par].at[k], sems.at[par]).wait()

    def compute(g, par):
        sl = pl.ds(g * L, L)
        isub = i_v[sl] & 7
        rows = lane + g * L
        accs = [a_v[sl] * c1v + c0v, ubv[sl] + ibv[sl],
                jnp.zeros((L,), jnp.float32), jnp.zeros((L,), jnp.float32)]
        dc = jnp.zeros((L,), jnp.int32)
        rbase = rows << 6
        for d in range(N_DIM):
            xu = plsc.load_gather(vu_hit, [rbase + dc])
            xi = plsc.load_gather(ibf[par], [lane, isub, dc])
            accs[d % 4] = accs[d % 4] + xu * xi
            dc = dc + 1
        vals_v[sl] = (accs[0] + accs[1]) + (accs[2] + accs[3])

    fire(0, 0)
    fire(1, 1)

    def gpair(p, carry):
        g0 = 2 * p
        idrain(0)
        compute(g0, 0)

        @pl.when(g0 + 2 < HGRP)
        def _():
            fire(g0 + 2, 0)

        idrain(1)
        compute(g0 + 1, 1)

        @pl.when(g0 + 3 < HGRP)
        def _():
            fire(g0 + 3, 1)

        return carry

    lax.fori_loop(0, HGRP // 2, gpair, 0)
    pltpu.sync_copy(ids_sw, ids_out.at[wid])
    pltpu.sync_copy(vals_v, vals_out.at[wid])


def _redist_body(ids_hbm, vals_hbm, out_hbm, ids_all, vals_all, out_v):
    wid = _wid()
    pltpu.sync_copy(ids_hbm, ids_all)
    pltpu.sync_copy(vals_hbm, vals_all)

    def w_iter(w, c):
        def q_iter(q, c2):
            sl = pl.ds(q * L, L)
            ids16 = ids_all[w, sl]
            m = (ids16 >> 9) == wid
            plsc.store_scatter(out_v, [ids16 & (PER_W - 1)],
                               vals_all[w, sl], mask=m)
            return c2

        return lax.fori_loop(0, HGRP, q_iter, c, unroll=2)

    lax.fori_loop(0, NW, w_iter, 0)
    pltpu.sync_copy(out_v, out_hbm.at[pl.ds(wid * PER_W, PER_W)])


def _mesh():
    return plsc.VectorSubcoreMesh(core_axis_name="c", subcore_axis_name="s")


@jax.jit
def _mfpoly2_sc(u1, ip, ap, user_vect_t, ub_flat, item_vect, ib_flat,
                c1, c0):
    cp = pltpu.CompilerParams(needs_layout_passes=False)
    f32, i32 = jnp.float32, jnp.int32
    main = functools.partial(
        pl.kernel, mesh=_mesh(), compiler_params=cp,
        out_type=(jax.ShapeDtypeStruct((NW, HITCAP), i32),
                  jax.ShapeDtypeStruct((NW, HITCAP), f32)),
        scratch_types=[
            pltpu.VMEM((UCHUNK,), i32),
            pltpu.VMEM((HITCAP,), i32),
            pltpu.VMEM((HITCAP,), i32),
            pltpu.VMEM((WCAP,), i32),
            pltpu.VMEM((WCAP,), i32),
            pltpu.VMEM((HITCAP,), i32),
            pltpu.VMEM((HITCAP,), i32),
            pltpu.VMEM((8, WIN), f32),
            pltpu.VMEM((8, WIN), f32),
            pltpu.VMEM((8, 64), f32),
            pltpu.VMEM((HITCAP * N_DIM,), f32),
            pltpu.VMEM((HITCAP,), i32),
            pltpu.VMEM((HITCAP,), f32),
            pltpu.VMEM((HITCAP,), f32),
            pltpu.VMEM((HITCAP,), f32),
            pltpu.VMEM((HITCAP,), f32),
            pltpu.VMEM((L,), f32),
            pltpu.VMEM((L,), f32),
            pltpu.VMEM((L, 8, N_DIM), f32),
            pltpu.VMEM((L, 8, N_DIM), f32),
            pltpu.SMEM((1,), i32),
            pltpu.SemaphoreType.DMA,
            pltpu.SemaphoreType.DMA((2,)),
        ])(_main_body)
    ids, vals = main(u1, user_vect_t, item_vect, ip, ap, ub_flat, ib_flat,
                     c1, c0)

    redist = functools.partial(
        pl.kernel, mesh=_mesh(), compiler_params=cp,
        out_type=jax.ShapeDtypeStruct((BATCH,), f32),
        scratch_types=[
            pltpu.VMEM((NW, HITCAP), i32),
            pltpu.VMEM((NW, HITCAP), f32),
            pltpu.VMEM((PER_W,), f32),
        ])(_redist_body)
    return redist(ids, vals)


def kernel(u, i, a, user_vect, user_bias, item_vect, item_bias, glob_bias,
           age1_w, age1_b, age2_w, age2_b):
    n = u.shape[0]
    u32 = u.astype(jnp.int32)
    i32_ = i.astype(jnp.int32)
    ip = jnp.pad(i32_, (0, L))
    ap = jnp.pad(a, (0, L))
    c1 = age1_w[0, 0] * age2_w[0, 0]
    c0 = glob_bias[0, 0] + age1_b[0] * age2_w[0, 0] + age2_b[0]
    c1v = jnp.full((L,), c1, jnp.float32)
    c0v = jnp.full((L,), c0, jnp.float32)
    out = _mfpoly2_sc(u32, ip, ap, user_vect.T, user_bias.reshape(-1),
                      item_vect, item_bias.reshape(-1), c1v, c0v)
    return out.reshape(n)


# speedup vs baseline: 1.1348x; 1.0609x over previous
import functools

import jax
import jax.numpy as jnp
from jax import lax
from jax.experimental import pallas as pl
from jax.experimental.pallas import tpu as pltpu
from jax.experimental.pallas import tpu_sc as plsc

BATCH = 16384
N_DIM = 64
N_USERS = 1000000
L = 16
NC, NS = 2, 16
NW = NC * NS
PER_W = BATCH // NW
WIN = 1024
NWIN = N_USERS // WIN
TAIL_LO = NWIN * WIN
TAIL_W = NWIN % NW
TAIL_N = N_USERS - TAIL_LO
HITCAP = 768
HGRP = HITCAP // L
WCAP = 96
UCHUNK = 2048
DUMMY = BATCH
HUGEU = 1 << 30


def _wid():
    return lax.axis_index("s") * NC + lax.axis_index("c")


def _popcount(m):
    return plsc.all_reduce_population_count(m)[0]


def _main_body(u_hbm, uvt_hbm, iv_hbm, ip_hbm, ap_hbm, ub_hbm, ib_hbm,
               c1_hbm, c0_hbm, ids_out, vals_out,
               u_ch, hit_ids, hit_us, chid, chu, ids_sw, us_sw,
               buf0, buf1, buf2, buf3, tbuf, vu_hit, i_v, a_v, ubv, ibv,
               vals_v,
               c1_v, c0_v, ibf0, ibf1, posm,
               bsem, sems):
    bufs = [buf0, buf1, buf2, buf3]
    ibf = [ibf0, ibf1]
    wid = _wid()
    lane = jnp.arange(L, dtype=jnp.int32)

    def fill(t, c):
        sl = pl.ds(t * L, L)
        hit_ids[sl] = jnp.full((L,), DUMMY, jnp.int32)
        hit_us[sl] = jnp.full((L,), HUGEU, jnp.int32)
        ids_sw[sl] = jnp.full((L,), DUMMY, jnp.int32)
        us_sw[sl] = jnp.zeros((L,), jnp.int32)
        return c

    lax.fori_loop(0, HGRP, fill, 0)

    pos = jnp.int32(0)
    for c in range(BATCH // UCHUNK):
        pltpu.sync_copy(u_hbm.at[pl.ds(c * UCHUNK, UCHUNK)], u_ch)

        def scan(t, p, c=c):
            uv = u_ch[pl.ds(t * L, L)]
            m = ((uv >> 10) & (NW - 1)) == wid
            plsc.store_compressed(hit_ids.at[pl.ds(p, L)],
                                  lane + (c * UCHUNK + t * L), mask=m)
            plsc.store_compressed(hit_us.at[pl.ds(p, L)], uv, mask=m)
            return p + _popcount(m)

        pos = lax.fori_loop(0, UCHUNK // L, scan, pos, unroll=2)

    def fetch(lo, g, par, width):
        coff = pl.multiple_of(lo, 128)
        pltpu.async_copy(
            uvt_hbm.at[pl.ds(8 * g, 8), pl.ds(coff, width)],
            bufs[par].at[:, pl.ds(0, width)],
            sems.at[par])

    def drain(par, width):
        pltpu.make_async_copy(
            uvt_hbm.at[pl.ds(0, 8), pl.ds(0, width)],
            bufs[par].at[:, pl.ds(0, width)],
            sems.at[par]).wait()

    def win_lo(w):
        return (wid + NW * w) * WIN

    def refill(lo, width):
        def rf(t, cpos):
            sl = pl.ds(t * L, L)
            uh = hit_us[sl]
            m = (uh >= lo) & (uh < lo + width)
            plsc.store_compressed(chu.at[pl.ds(cpos, L)], uh, mask=m)
            plsc.store_compressed(chid.at[pl.ds(cpos, L)], hit_ids[sl],
                                  mask=m)
            return cpos + _popcount(m)

        cnt = lax.fori_loop(0, HGRP, rf, 0, unroll=2)
        p2 = posm[0]

        def sw(t, c):
            m = lane < (cnt - t * L)
            sl = pl.ds(t * L, L)
            rows = p2 + t * L + lane
            plsc.store_scatter(ids_sw, [rows], chid[sl], mask=m)
            plsc.store_scatter(us_sw, [rows], chu[sl], mask=m)
            return c

        lax.fori_loop(0, (cnt + L - 1) >> 4, sw, 0)
        return cnt

    def extract(buf, lo, g, cnt):
        p2 = posm[0]

        def xgrp(t, c):
            m = lane < (cnt - t * L)
            sl = pl.ds(t * L, L)
            uoff = chu[sl] - lo
            rbase = (p2 + t * L + lane) << 6
            dl = jnp.full((L,), 0, jnp.int32)
            dg = jnp.full((L,), 8 * g, jnp.int32)
            for _ in range(8):
                xd = plsc.load_gather(buf, [dl, uoff], mask=m)
                plsc.store_scatter(vu_hit, [rbase + dg], xd, mask=m)
                dl = dl + 1
                dg = dg + 1
            return c

        lax.fori_loop(0, (cnt + L - 1) >> 4, xgrp, 0)

    nwin = jnp.where(wid < (NWIN % NW), NWIN // NW + 1,
                     NWIN // NW).astype(jnp.int32)
    posm[0] = 0

    fetch(win_lo(0), 0, 0, WIN)
    fetch(win_lo(0), 1, 1, WIN)
    fetch(win_lo(0), 2, 2, WIN)

    def wloop(w, carry):
        cnt = refill(win_lo(w), WIN)
        for g in range(8):
            par = g % 4
            nxt = (g + 3) % 4
            if g < 5:
                fetch(win_lo(w), g + 3, nxt, WIN)
            else:
                @pl.when(w + 1 < nwin)
                def _():
                    fetch(win_lo(w + 1), g - 5, nxt, WIN)
            drain(par, WIN)
            extract(bufs[par], win_lo(w), g, cnt)
        posm[0] = posm[0] + cnt
        return carry

    lax.fori_loop(0, nwin, wloop, 0)

    @pl.when(wid == TAIL_W)
    def _():
        cnt = refill(TAIL_LO, 512)
        for g in range(8):
            par = g % 4
            pltpu.async_copy(
                uvt_hbm.at[pl.ds(8 * g, 8), pl.ds(TAIL_LO, 512)],
                bufs[par].at[:, pl.ds(0, 512)], sems.at[par])
            drain(par, 512)
            extract(bufs[par], TAIL_LO, g, cnt)
        posm[0] = posm[0] + cnt
        cnt2 = refill(TAIL_LO + 512, 64)
        for g in range(8):
            pltpu.async_copy(
                uvt_hbm.at[pl.ds(8 * g, 8), pl.ds(TAIL_LO + 512, 64)],
                tbuf, sems.at[0])
            pltpu.make_async_copy(
                uvt_hbm.at[pl.ds(8 * g, 8), pl.ds(TAIL_LO + 512, 64)],
                tbuf, sems.at[0]).wait()
            extract(tbuf, TAIL_LO + 512, g, cnt2)
        posm[0] = posm[0] + cnt2

    d1 = [pltpu.async_copy(t_hbm.at[ids_sw.at[pl.ds(c * 128, 128)]],
                           t_v.at[pl.ds(c * 128, 128)], bsem)
          for c in range(HITCAP // 128)
          for (t_hbm, t_v) in ((ip_hbm, i_v), (ap_hbm, a_v))]
    d1.append(pltpu.async_copy(c1_hbm, c1_v, bsem))
    d1.append(pltpu.async_copy(c0_hbm, c0_v, bsem))
    for d in d1:
        d.wait()
    d2 = [pltpu.async_copy(t_hbm.at[t_idx.at[pl.ds(c * 128, 128)]],
                           t_v.at[pl.ds(c * 128, 128)], bsem)
          for c in range(HITCAP // 128)
          for (t_hbm, t_idx, t_v) in ((ub_hbm, us_sw, ubv),
                                      (ib_hbm, i_v, ibv))]
    for d in d2:
        d.wait()

    c1v = c1_v[...]
    c0v = c0_v[...]

    def fire(g, par):
        ivals = i_v[pl.ds(g * L, L)]
        for k in range(L):
            is_ = pl.multiple_of((ivals[k] >> 3) << 3, 8)
            pltpu.async_copy(iv_hbm.at[pl.ds(is_, 8), :], ibf[par].at[k],
                             sems.at[par])

    def idrain(par):
        for k in range(L):
            pltpu.make_async_copy(iv_hbm.at[pl.ds(0, 8), :],
                                  ibf[par].at[k], sems.at[par]).wait()

    def compute(g, par):
        sl = pl.ds(g * L, L)
        isub = i_v[sl] & 7
        rows = lane + g * L
        accs = [a_v[sl] * c1v + c0v, ubv[sl] + ibv[sl],
                jnp.zeros((L,), jnp.float32), jnp.zeros((L,), jnp.float32)]
        dc = jnp.zeros((L,), jnp.int32)
        rbase = rows << 6
        for d in range(N_DIM):
            xu = plsc.load_gather(vu_hit, [rbase + dc])
            xi = plsc.load_gather(ibf[par], [lane, isub, dc])
            accs[d % 4] = accs[d % 4] + xu * xi
            dc = dc + 1
        vals_v[sl] = (accs[0] + accs[1]) + (accs[2] + accs[3])

    fire(0, 0)
    fire(1, 1)

    def gpair(p, carry):
        g0 = 2 * p
        idrain(0)
        compute(g0, 0)

        @pl.when(g0 + 2 < HGRP)
        def _():
            fire(g0 + 2, 0)

        idrain(1)
        compute(g0 + 1, 1)

        @pl.when(g0 + 3 < HGRP)
        def _():
            fire(g0 + 3, 1)

        return carry

    lax.fori_loop(0, HGRP // 2, gpair, 0)
    pltpu.sync_copy(ids_sw, ids_out.at[wid])
    pltpu.sync_copy(vals_v, vals_out.at[wid])


def _redist_body(ids_hbm, vals_hbm, out_hbm, ids_all, vals_all, out_v):
    wid = _wid()
    pltpu.sync_copy(ids_hbm, ids_all)
    pltpu.sync_copy(vals_hbm, vals_all)

    def w_iter(w, c):
        def q_iter(q, c2):
            sl = pl.ds(q * L, L)
            ids16 = ids_all[w, sl]
            m = (ids16 >> 9) == wid
            plsc.store_scatter(out_v, [ids16 & (PER_W - 1)],
                               vals_all[w, sl], mask=m)
            return c2

        return lax.fori_loop(0, HGRP, q_iter, c, unroll=2)

    lax.fori_loop(0, NW, w_iter, 0)
    pltpu.sync_copy(out_v, out_hbm.at[pl.ds(wid * PER_W, PER_W)])


def _mesh():
    return plsc.VectorSubcoreMesh(core_axis_name="c", subcore_axis_name="s")


@jax.jit
def _mfpoly2_sc(u1, ip, ap, user_vect_t, ub_flat, item_vect, ib_flat,
                c1, c0):
    cp = pltpu.CompilerParams(needs_layout_passes=False)
    f32, i32 = jnp.float32, jnp.int32
    main = functools.partial(
        pl.kernel, mesh=_mesh(), compiler_params=cp,
        out_type=(jax.ShapeDtypeStruct((NW, HITCAP), i32),
                  jax.ShapeDtypeStruct((NW, HITCAP), f32)),
        scratch_types=[
            pltpu.VMEM((UCHUNK,), i32),
            pltpu.VMEM((HITCAP,), i32),
            pltpu.VMEM((HITCAP,), i32),
            pltpu.VMEM((WCAP,), i32),
            pltpu.VMEM((WCAP,), i32),
            pltpu.VMEM((HITCAP,), i32),
            pltpu.VMEM((HITCAP,), i32),
            pltpu.VMEM((8, WIN), f32),
            pltpu.VMEM((8, WIN), f32),
            pltpu.VMEM((8, WIN), f32),
            pltpu.VMEM((8, WIN), f32),
            pltpu.VMEM((8, 64), f32),
            pltpu.VMEM((HITCAP * N_DIM,), f32),
            pltpu.VMEM((HITCAP,), i32),
            pltpu.VMEM((HITCAP,), f32),
            pltpu.VMEM((HITCAP,), f32),
            pltpu.VMEM((HITCAP,), f32),
            pltpu.VMEM((HITCAP,), f32),
            pltpu.VMEM((L,), f32),
            pltpu.VMEM((L,), f32),
            pltpu.VMEM((L, 8, N_DIM), f32),
            pltpu.VMEM((L, 8, N_DIM), f32),
            pltpu.SMEM((1,), i32),
            pltpu.SemaphoreType.DMA,
            pltpu.SemaphoreType.DMA((2,)),
        ])(_main_body)
    ids, vals = main(u1, user_vect_t, item_vect, ip, ap, ub_flat, ib_flat,
                     c1, c0)

    redist = functools.partial(
        pl.kernel, mesh=_mesh(), compiler_params=cp,
        out_type=jax.ShapeDtypeStruct((BATCH,), f32),
        scratch_types=[
            pltpu.VMEM((NW, HITCAP), i32),
            pltpu.VMEM((NW, HITCAP), f32),
            pltpu.VMEM((PER_W,), f32),
        ])(_redist_body)
    return redist(ids, vals)


def kernel(u, i, a, user_vect, user_bias, item_vect, item_bias, glob_bias,
           age1_w, age1_b, age2_w, age2_b):
    n = u.shape[0]
    u32 = u.astype(jnp.int32)
    i32_ = i.astype(jnp.int32)
    ip = jnp.pad(i32_, (0, L))
    ap = jnp.pad(a, (0, L))
    c1 = age1_w[0, 0] * age2_w[0, 0]
    c0 = glob_bias[0, 0] + age1_b[0] * age2_w[0, 0] + age2_b[0]
    c1v = jnp.full((L,), c1, jnp.float32)
    c0v = jnp.full((L,), c0, jnp.float32)
    out = _mfpoly2_sc(u32, ip, ap, user_vect.T, user_bias.reshape(-1),
                      item_vect, item_bias.reshape(-1), c1v, c0v)
    return out.reshape(n)


# speedup vs baseline: 1.4987x; 1.3207x over previous
import functools

import jax
import jax.numpy as jnp
from jax import lax
from jax.experimental import pallas as pl
from jax.experimental.pallas import tpu as pltpu
from jax.experimental.pallas import tpu_sc as plsc

BATCH = 16384
N_DIM = 64
L = 16
NC, NS = 2, 16
NW = NC * NS
PER_W = BATCH // NW
GROUPS = PER_W // L
BCHUNK = 128


def _body(u_hbm, i_hbm, a_hbm, uv_hbm, ub_hbm, iv_hbm, ib_hbm,
          c1_hbm, c0_hbm, out_hbm,
          u_v, i_v, a_v, c1_v, c0_v,
          ubf0, ubf1, ibf0, ibf1, ubias, ibias, out_v,
          bsem, sems):
    ubf = [ubf0, ubf1]
    ibf = [ibf0, ibf1]
    wid = lax.axis_index("s") * NC + lax.axis_index("c")
    base = wid * PER_W

    pltpu.sync_copy(u_hbm.at[pl.ds(base, PER_W)], u_v)
    pltpu.sync_copy(i_hbm.at[pl.ds(base, PER_W)], i_v)
    pltpu.sync_copy(a_hbm.at[pl.ds(base, PER_W)], a_v)
    pltpu.sync_copy(c1_hbm, c1_v)
    pltpu.sync_copy(c0_hbm, c0_v)

    bias_descs = [
        pltpu.async_copy(t_hbm.at[t_v.at[pl.ds(c * BCHUNK, BCHUNK)]],
                         t_bias.at[pl.ds(c * BCHUNK, BCHUNK)], bsem)
        for c in range(PER_W // BCHUNK)
        for (t_hbm, t_v, t_bias) in ((ub_hbm, u_v, ubias),
                                     (ib_hbm, i_v, ibias))
    ]

    c1v = c1_v[...]
    c0v = c0_v[...]
    lane = jnp.arange(L, dtype=jnp.int32)

    def fire(g, par):
        uvals = u_v[pl.ds(g * L, L)]
        ivals = i_v[pl.ds(g * L, L)]
        for k in range(L):
            us = pl.multiple_of((uvals[k] >> 3) << 3, 8)
            pltpu.async_copy(uv_hbm.at[pl.ds(us, 8), :], ubf[par].at[k],
                             sems.at[par])
            is_ = pl.multiple_of((ivals[k] >> 3) << 3, 8)
            pltpu.async_copy(iv_hbm.at[pl.ds(is_, 8), :], ibf[par].at[k],
                             sems.at[par])

    def drain(par):
        for k in range(L):
            pltpu.make_async_copy(uv_hbm.at[pl.ds(0, 8), :],
                                  ubf[par].at[k], sems.at[par]).wait()
            pltpu.make_async_copy(uv_hbm.at[pl.ds(0, 8), :],
                                  ibf[par].at[k], sems.at[par]).wait()

    def compute(g, par):
        sl = pl.ds(g * L, L)
        usub = u_v[sl] & 7
        isub = i_v[sl] & 7
        accs = [a_v[sl] * c1v + c0v,
                ubias[sl] + ibias[sl],
                jnp.zeros((L,), jnp.float32),
                jnp.zeros((L,), jnp.float32)]
        dc = jnp.zeros((L,), jnp.int32)
        for d in range(N_DIM):
            xu = plsc.load_gather(ubf[par], [lane, usub, dc])
            xi = plsc.load_gather(ibf[par], [lane, isub, dc])
            accs[d % 4] = accs[d % 4] + xu * xi
            dc = dc + 1
        out_v[sl] = (accs[0] + accs[1]) + (accs[2] + accs[3])

    fire(0, 0)
    fire(1, 1)
    for d in bias_descs:
        d.wait()

    def pair(gp, carry):
        g0 = gp * 2
        drain(0)
        compute(g0, 0)

        @pl.when(g0 + 2 < GROUPS)
        def _():
            fire(g0 + 2, 0)

        drain(1)
        compute(g0 + 1, 1)

        @pl.when(g0 + 3 < GROUPS)
        def _():
            fire(g0 + 3, 1)

        return carry

    lax.fori_loop(0, GROUPS // 2, pair, 0)

    pltpu.sync_copy(out_v, out_hbm.at[pl.ds(base, PER_W)])


@jax.jit
def _mfpoly2_sc(u1, i1, a1, user_vect, ub_flat, item_vect, ib_flat, c1, c0):
    mesh = plsc.VectorSubcoreMesh(core_axis_name="c", subcore_axis_name="s")
    f = functools.partial(
        pl.kernel,
        mesh=mesh,
        compiler_params=pltpu.CompilerParams(needs_layout_passes=False),
        out_type=jax.ShapeDtypeStruct((BATCH,), jnp.float32),
        scratch_types=[
            pltpu.VMEM((PER_W,), jnp.int32),
            pltpu.VMEM((PER_W,), jnp.int32),
            pltpu.VMEM((PER_W,), jnp.float32),
            pltpu.VMEM((L,), jnp.float32),
            pltpu.VMEM((L,), jnp.float32),
            pltpu.VMEM((L, 8, N_DIM), jnp.float32),
            pltpu.VMEM((L, 8, N_DIM), jnp.float32),
            pltpu.VMEM((L, 8, N_DIM), jnp.float32),
            pltpu.VMEM((L, 8, N_DIM), jnp.float32),
            pltpu.VMEM((PER_W,), jnp.float32),
            pltpu.VMEM((PER_W,), jnp.float32),
            pltpu.VMEM((PER_W,), jnp.float32),
            pltpu.SemaphoreType.DMA,
            pltpu.SemaphoreType.DMA((2,)),
        ],
    )(_body)
    return f(u1, i1, a1, user_vect, ub_flat, item_vect, ib_flat, c1, c0)


def kernel(u, i, a, user_vect, user_bias, item_vect, item_bias, glob_bias,
           age1_w, age1_b, age2_w, age2_b):
    n = u.shape[0]
    c1 = age1_w[0, 0] * age2_w[0, 0]
    c0 = glob_bias[0, 0] + age1_b[0] * age2_w[0, 0] + age2_b[0]
    c1v = jnp.full((L,), c1, jnp.float32)
    c0v = jnp.full((L,), c0, jnp.float32)
    return _mfpoly2_sc(u.astype(jnp.int32), i.astype(jnp.int32), a,
                       user_vect, user_bias.T.reshape(-1),
                       item_vect, item_bias.T.reshape(-1), c1v, c0v)


# speedup vs baseline: 1.5036x; 1.0033x over previous
import functools

import jax
import jax.numpy as jnp
from jax import lax
from jax.experimental import pallas as pl
from jax.experimental.pallas import tpu as pltpu
from jax.experimental.pallas import tpu_sc as plsc

BATCH = 16384
N_DIM = 64
L = 16
NC, NS = 2, 16
NW = NC * NS
PER_W = BATCH // NW
GROUPS = PER_W // L
BCHUNK = 128


def _body(u_hbm, i_hbm, a_hbm, uv_hbm, ub_hbm, iv_hbm, ib_hbm,
          c1_hbm, c0_hbm, out_hbm,
          u_v, i_v, a_v, c1_v, c0_v,
          ubf0, ubf1, ibf0, ibf1, ubias, ibias, out_v,
          bsem, sems):
    ubf = [ubf0, ubf1]
    ibf = [ibf0, ibf1]
    wid = lax.axis_index("s") * NC + lax.axis_index("c")
    base = wid * PER_W

    pltpu.sync_copy(u_hbm.at[pl.ds(base, PER_W)], u_v)
    pltpu.sync_copy(i_hbm.at[pl.ds(base, PER_W)], i_v)
    pltpu.sync_copy(a_hbm.at[pl.ds(base, PER_W)], a_v)
    pltpu.sync_copy(c1_hbm, c1_v)
    pltpu.sync_copy(c0_hbm, c0_v)

    bias_descs = [
        pltpu.async_copy(t_hbm.at[t_v.at[pl.ds(c * BCHUNK, BCHUNK)]],
                         t_bias.at[pl.ds(c * BCHUNK, BCHUNK)], bsem)
        for c in range(PER_W // BCHUNK)
        for (t_hbm, t_v, t_bias) in ((ub_hbm, u_v, ubias),
                                     (ib_hbm, i_v, ibias))
    ]

    c1v = c1_v[...]
    c0v = c0_v[...]
    lane = jnp.arange(L, dtype=jnp.int32)

    def fire(g, par):
        uvals = u_v[pl.ds(g * L, L)]
        ivals = i_v[pl.ds(g * L, L)]
        for k in range(L):
            us = pl.multiple_of((uvals[k] >> 3) << 3, 8)
            pltpu.async_copy(uv_hbm.at[pl.ds(us, 8), :], ubf[par].at[k],
                             sems.at[par])
            is_ = pl.multiple_of((ivals[k] >> 3) << 3, 8)
            pltpu.async_copy(iv_hbm.at[pl.ds(is_, 8), :], ibf[par].at[k],
                             sems.at[par])

    def drain(par):
        for k in range(L):
            pltpu.make_async_copy(uv_hbm.at[pl.ds(0, 8), :],
                                  ubf[par].at[k], sems.at[par]).wait()
            pltpu.make_async_copy(uv_hbm.at[pl.ds(0, 8), :],
                                  ibf[par].at[k], sems.at[par]).wait()

    def compute(g, par):
        sl = pl.ds(g * L, L)
        usub = u_v[sl] & 7
        isub = i_v[sl] & 7
        accs = [a_v[sl] * c1v + c0v,
                ubias[sl] + ibias[sl],
                jnp.zeros((L,), jnp.float32),
                jnp.zeros((L,), jnp.float32)]
        dc = jnp.zeros((L,), jnp.int32)
        for d in range(N_DIM):
            xu = plsc.load_gather(ubf[par], [lane, usub, dc])
            xi = plsc.load_gather(ibf[par], [lane, isub, dc])
            accs[d % 4] = accs[d % 4] + xu * xi
            dc = dc + 1
        out_v[sl] = (accs[0] + accs[1]) + (accs[2] + accs[3])

    fire(0, 0)
    fire(1, 1)
    for d in bias_descs:
        d.wait()

    def pair(gp, carry):
        g0 = gp * 2
        drain(0)
        compute(g0, 0)

        @pl.when(g0 + 2 < GROUPS)
        def _():
            fire(g0 + 2, 0)

        drain(1)
        compute(g0 + 1, 1)

        @pl.when(g0 + 3 < GROUPS)
        def _():
            fire(g0 + 3, 1)

        return carry

    lax.fori_loop(0, GROUPS // 2, pair, 0)

    pltpu.sync_copy(out_v, out_hbm.at[pl.ds(base, PER_W)])


@jax.jit
def _mfpoly2_sc(u1, i1, a1, user_vect, ub_flat, item_vect, ib_flat, c1, c0):
    mesh = plsc.VectorSubcoreMesh(core_axis_name="c", subcore_axis_name="s")
    f = functools.partial(
        pl.kernel,
        mesh=mesh,
        compiler_params=pltpu.CompilerParams(needs_layout_passes=False),
        out_type=jax.ShapeDtypeStruct((BATCH,), jnp.float32),
        scratch_types=[
            pltpu.VMEM((PER_W,), jnp.int32),
            pltpu.VMEM((PER_W,), jnp.int32),
            pltpu.VMEM((PER_W,), jnp.float32),
            pltpu.VMEM((L,), jnp.float32),
            pltpu.VMEM((L,), jnp.float32),
            pltpu.VMEM((L, 8, N_DIM), jnp.float32),
            pltpu.VMEM((L, 8, N_DIM), jnp.float32),
            pltpu.VMEM((L, 8, N_DIM), jnp.float32),
            pltpu.VMEM((L, 8, N_DIM), jnp.float32),
            pltpu.VMEM((PER_W,), jnp.float32),
            pltpu.VMEM((PER_W,), jnp.float32),
            pltpu.VMEM((PER_W,), jnp.float32),
            pltpu.SemaphoreType.DMA,
            pltpu.SemaphoreType.DMA((2,)),
        ],
    )(_body)
    return f(u1, i1, a1, user_vect, ub_flat, item_vect, ib_flat, c1, c0)


def kernel(u, i, a, user_vect, user_bias, item_vect, item_bias, glob_bias,
           age1_w, age1_b, age2_w, age2_b):
    n = u.shape[0]
    c1 = age1_w[0, 0] * age2_w[0, 0]
    c0 = glob_bias[0, 0] + age1_b[0] * age2_w[0, 0] + age2_b[0]
    c1v = jnp.full((L,), c1, jnp.float32)
    c0v = jnp.full((L,), c0, jnp.float32)
    out = _mfpoly2_sc(u.astype(jnp.int32), i.astype(jnp.int32), a,
                      user_vect, user_bias.T.reshape(-1),
                      item_vect, item_bias.T.reshape(-1), c1v, c0v)
    nudge = jnp.take(user_vect, u[:1], axis=0).sum() * 0.0
    return out + nudge
